# Initial kernel scaffold; baseline (speedup 1.0000x reference)
import jax
import jax.numpy as jnp
from jax.experimental import pallas as pl


def kernel(in_item_id, seqlen, item_table, w_h, w_hf, agnn_q, agnn_k, agnn_v, agnn_ffn_w, agnn_ffn_b, fuse_w, fuse_b, att1_w, att1_b, att2_w, att2_b, att3_w, att4_w, att4_b):
    raise NotImplementedError("write your pallas kernel here")


# speedup vs baseline: 4.8419x; 4.8419x over previous
import jax
import jax.numpy as jnp
from jax.experimental import pallas as pl

_interpret = False

H = 4


def _graph_body(src_ref, dst_ref, pos_ref, seq_ref, tok_ref, adj_ref, pres_ref):
    i = pl.program_id(0)
    eb = src_ref.shape[0]
    tb = tok_ref.shape[0]
    n = adj_ref.shape[0]
    iota_e = jax.lax.broadcasted_iota(jnp.int32, (eb, n), 1)
    emask = pos_ref[...] < (seq_ref[...] - 1)
    srcoh = ((src_ref[...] == iota_e) & emask).astype(jnp.bfloat16)
    dstoh = (dst_ref[...] == iota_e).astype(jnp.bfloat16)
    upd = jax.lax.dot_general(dstoh, srcoh, (((0,), (0,)), ((), ())),
                              preferred_element_type=jnp.float32)
    iota_t = jax.lax.broadcasted_iota(jnp.int32, (tb, n), 1)
    p = jnp.max((tok_ref[...] == iota_t).astype(jnp.float32), axis=0,
                keepdims=True)

    @pl.when(i == 0)
    def _():
        adj_ref[...] = jnp.zeros_like(adj_ref)
        pres_ref[...] = jnp.zeros_like(pres_ref)

    adj_ref[...] += upd
    pres_ref[...] = jnp.maximum(pres_ref[...], jnp.broadcast_to(p, pres_ref.shape))


def _dense_body(adj_ref, pres_ref, itab_ref, wh_ref, whf_ref, aq_ref, ak_ref,
                av_ref, afw_ref, afb_ref, fw_ref, fb_ref, a1w_ref, a1b_ref,
                a2w_ref, a2b_ref, tthl_ref):
    f32 = jnp.float32
    adj = adj_ref[...]
    pres = pres_ref[...]
    itab = itab_ref[...]
    n, d = itab.shape
    ones_col = jnp.ones((n, 1), f32)
    in_deg = jnp.dot(adj, ones_col, preferred_element_type=f32)
    out_deg = jax.lax.dot_general(adj, ones_col, (((0,), (0,)), ((), ())),
                                  preferred_element_type=f32)
    ns = jax.lax.rsqrt(jnp.maximum(out_deg, 1.0))
    nd = jax.lax.rsqrt(jnp.maximum(in_deg, 1.0))

    wh = wh_ref[...]
    whf = whf_ref[...]

    def gnn(hid):
        hd = jnp.dot(hid, wh, preferred_element_type=f32)
        h1, h2, h3 = hd[:, :d], hd[:, d:2 * d], hd[:, 2 * d:]
        agg = nd * jnp.dot(adj, h1 * ns, preferred_element_type=f32)
        hf = jnp.dot(agg, whf, preferred_element_type=f32)
        f1, f2 = hf[:, :d], hf[:, d:]
        return h3 + jnp.maximum(f1 + h2, 0.0) * f2

    hid = itab
    for _ in range(aq_ref.shape[0]):
        hid = gnn(hid)

    dh = d // H
    x = itab
    for i in range(aq_ref.shape[0]):
        q = jnp.dot(x, aq_ref[i], preferred_element_type=f32)
        k = jnp.dot(x, ak_ref[i], preferred_element_type=f32)
        v = jnp.dot(x, av_ref[i], preferred_element_type=f32) * pres
        cols = []
        for h in range(H):
            qh = q[:, h * dh:(h + 1) * dh]
            kh = k[:, h * dh:(h + 1) * dh]
            vh = v[:, h * dh:(h + 1) * dh]
            sc = jnp.tanh(jax.lax.dot_general(
                qh, kh, (((1,), (1,)), ((), ())), preferred_element_type=f32))
            cols.append(jnp.dot(sc, vh, preferred_element_type=f32))
        att = jnp.concatenate(cols, axis=1)
        att = jnp.maximum(jnp.dot(att, afw_ref[i], preferred_element_type=f32)
                          + afb_ref[i:i + 1], 0.0)
        x = x + att

    fw = fw_ref[...]
    t_tab = (jnp.dot(hid, fw[:d], preferred_element_type=f32)
             + jnp.dot(x, fw[d:], preferred_element_type=f32) + fb_ref[...])
    t2_tab = (jnp.dot(t_tab, a2w_ref[...], preferred_element_type=f32)
              + a2b_ref[...])
    t2h = t2_tab.astype(jnp.bfloat16)
    th = t_tab.astype(jnp.bfloat16)
    t2l = (t2_tab - t2h.astype(f32)).astype(jnp.bfloat16)
    tl = (t_tab - th.astype(f32)).astype(jnp.bfloat16)
    tthl_ref[...] = jnp.concatenate([t2h, th, t2l, tl], axis=1)


def _token_body(tok_ref, seq_ref, tthl_ref, a1w_ref, a1b_ref, a3r_ref,
                a4w_ref, a4b_ref, out_ref):
    f32 = jnp.float32
    tb = tok_ref.shape[0]
    n = tthl_ref.shape[0]
    rb, d = out_ref.shape
    ll = tb // rb
    iota_t = jax.lax.broadcasted_iota(jnp.int32, (tb, n), 1)
    oh = (tok_ref[...] == iota_t).astype(jnp.bfloat16)
    g = jnp.dot(oh, tthl_ref[...], preferred_element_type=f32)
    t2g = (g[:, :d] + g[:, 2 * d:3 * d]).reshape(rb, ll, d)
    tg = (g[:, d:2 * d] + g[:, 3 * d:]).reshape(rb, ll, d)
    mask = (tok_ref[...] != 0).astype(f32).reshape(rb, ll)

    lio = jax.lax.broadcasted_iota(jnp.int32, (rb, ll), 1)
    lsel = (lio == (seq_ref[...] - 1)).astype(f32)
    ht = jnp.sum(lsel[:, :, None] * tg, axis=1)
    q1 = jnp.dot(ht, a1w_ref[...], preferred_element_type=f32) + a1b_ref[...]
    sig = jax.nn.sigmoid(q1[:, None, :] + t2g)
    alpha = jnp.sum(sig * a3r_ref[...][None], axis=2)
    w = alpha * mask
    a = jnp.sum(w[:, :, None] * tg, axis=1)
    a4w = a4w_ref[...]
    out_ref[...] = (jnp.dot(a, a4w[:d], preferred_element_type=f32)
                    + jnp.dot(ht, a4w[d:], preferred_element_type=f32)
                    + a4b_ref[...])


def kernel(in_item_id, seqlen, item_table, w_h, w_hf, agnn_q, agnn_k, agnn_v,
           agnn_ffn_w, agnn_ffn_b, fuse_w, fuse_b, att1_w, att1_b, att2_w,
           att2_b, att3_w, att4_w, att4_b):
    f32 = jnp.float32
    b, l = in_item_id.shape
    n, d = item_table.shape
    ids = in_item_id.astype(jnp.int32)
    sl = seqlen.astype(jnp.int32)

    rb = 64
    nb = b // rb
    eb = rb * (l - 1)
    tb = rb * l

    srcf = ids[:, :-1].reshape(-1, 1)
    dstf = ids[:, 1:].reshape(-1, 1)
    posf = jnp.broadcast_to(jnp.arange(l - 1, dtype=jnp.int32)[None],
                            (b, l - 1)).reshape(-1, 1)
    seqf = jnp.broadcast_to(sl[:, None], (b, l - 1)).reshape(-1, 1)
    tokf = ids.reshape(-1, 1)

    adj, pres8 = pl.pallas_call(
        _graph_body,
        grid=(nb,),
        in_specs=[
            pl.BlockSpec((eb, 1), lambda i: (i, 0)),
            pl.BlockSpec((eb, 1), lambda i: (i, 0)),
            pl.BlockSpec((eb, 1), lambda i: (i, 0)),
            pl.BlockSpec((eb, 1), lambda i: (i, 0)),
            pl.BlockSpec((tb, 1), lambda i: (i, 0)),
        ],
        out_specs=[
            pl.BlockSpec((n, n), lambda i: (0, 0)),
            pl.BlockSpec((8, n), lambda i: (0, 0)),
        ],
        out_shape=[
            jax.ShapeDtypeStruct((n, n), f32),
            jax.ShapeDtypeStruct((8, n), f32),
        ],
        interpret=_interpret,
    )(srcf, dstf, posf, seqf, tokf)

    pres_col = pres8[0:1, :].T

    tthl = pl.pallas_call(
        _dense_body,
        out_shape=jax.ShapeDtypeStruct((n, 4 * d), jnp.bfloat16),
        interpret=_interpret,
    )(adj, pres_col, item_table, w_h, w_hf, agnn_q, agnn_k, agnn_v,
      agnn_ffn_w, agnn_ffn_b, fuse_w, fuse_b.reshape(1, d),
      att1_w, att1_b.reshape(1, d), att2_w, att2_b.reshape(1, d))

    out = pl.pallas_call(
        _token_body,
        grid=(nb,),
        in_specs=[
            pl.BlockSpec((tb, 1), lambda i: (i, 0)),
            pl.BlockSpec((rb, 1), lambda i: (i, 0)),
            pl.BlockSpec((n, 4 * d), lambda i: (0, 0)),
            pl.BlockSpec((d, d), lambda i: (0, 0)),
            pl.BlockSpec((1, d), lambda i: (0, 0)),
            pl.BlockSpec((1, d), lambda i: (0, 0)),
            pl.BlockSpec((2 * d, d), lambda i: (0, 0)),
            pl.BlockSpec((1, d), lambda i: (0, 0)),
        ],
        out_specs=pl.BlockSpec((rb, d), lambda i: (i, 0)),
        out_shape=jax.ShapeDtypeStruct((b, d), f32),
        interpret=_interpret,
    )(tokf, sl[:, None], tthl, att1_w, att1_b.reshape(1, d),
      att3_w.reshape(1, d), att4_w, att4_b.reshape(1, d))

    return out


# speedup vs baseline: 13.0808x; 2.7016x over previous
import functools

import jax
import jax.numpy as jnp
from jax import lax
from jax.experimental import pallas as pl
from jax.experimental.pallas import tpu as pltpu
from jax.experimental.pallas import tpu_sc as plsc

_interpret = False

H = 4

_NV = 1000
_ADJ_PAD = 1024000
_ROWS_W = 128
_TOK_W = _ROWS_W * 50


def _sc_graph_body(ids_hbm, idsf_hbm, posf_hbm, seqf_hbm, adj_out, pres_out,
                   tokf, tok2, posv, seqv, eidx2, val2, onesb, zbuf,
                   sem_a, sem_p, adj_sp, pres_sp):
    c = lax.axis_index("c")
    s = lax.axis_index("s")
    wid = c * 16 + s
    f32 = jnp.float32
    i32 = jnp.int32

    if True:
        def zfill(i, _):
            zbuf[pl.ds(i * 16, 16)] = jnp.zeros((16,), f32)
            return 0
        lax.fori_loop(0, 500, zfill, 0)
        for m in range(8):
            pltpu.sync_copy(zbuf.at[pl.ds(0, 8000)],
                            adj_sp.at[pl.ds(s * 64000 + m * 8000, 8000)])

        @pl.when(s == 0)
        def _():
            pltpu.sync_copy(zbuf.at[pl.ds(0, 1024)], pres_sp.at[pl.ds(0, 1024)])

        pltpu.sync_copy(ids_hbm.at[wid], tok2)
        pltpu.sync_copy(idsf_hbm.at[pl.ds(wid * _TOK_W, _TOK_W)],
                        tokf.at[pl.ds(0, _TOK_W)])
        pltpu.sync_copy(posf_hbm.at[pl.ds(wid * _TOK_W, _TOK_W)], posv)
        pltpu.sync_copy(seqf_hbm.at[pl.ds(wid * _TOK_W, _TOK_W)], seqv)
        tokf[pl.ds(_TOK_W, 16)] = jnp.zeros((16,), i32)

        def erow(j, _):
            base = j * 128
            for k in range(8):
                off = base + k * 16
                valid = posv[pl.ds(off, 16)] < (seqv[pl.ds(off, 16)] - 1)
                tok_v = tokf[pl.ds(off, 16)]
                tok_n = tokf[pl.ds(off + 1, 16)]
                eidx2[j, pl.ds(k * 16, 16)] = tok_n * _NV + tok_v
                val2[j, pl.ds(k * 16, 16)] = jnp.where(
                    valid, jnp.ones((16,), f32), jnp.zeros((16,), f32))
                onesb[j, pl.ds(k * 16, 16)] = jnp.ones((16,), f32)
            return 0
        lax.fori_loop(0, 50, erow, 0)

        plsc.subcore_barrier()
        adj_descs = [
            pltpu.make_async_copy(val2.at[j], adj_sp.at[eidx2.at[j]], sem_a)
            for j in range(50)
        ]
        pres_descs = [
            pltpu.make_async_copy(onesb.at[j], pres_sp.at[tok2.at[j]], sem_p)
            for j in range(50)
        ]
        for dsc in adj_descs:
            dsc.start(add=True)
        for dsc in pres_descs:
            dsc.start(add=True)
        for dsc in adj_descs:
            dsc.wait()
        for dsc in pres_descs:
            dsc.wait()
        plsc.subcore_barrier()

        pltpu.sync_copy(adj_sp.at[pl.ds(s * 64000, 64000)],
                        adj_out.at[c, pl.ds(s * 64000, 64000)])

        @pl.when(s == 0)
        def _():
            pltpu.sync_copy(pres_sp.at[pl.ds(0, 1024)], pres_out.at[c])


def _dense_body(adj0_ref, adj1_ref, pres_ref, itab_ref, wh_ref, whf_ref, aq_ref, ak_ref,
                av_ref, afw_ref, afb_ref, fw_ref, fb_ref, a1w_ref, a1b_ref,
                a2w_ref, a2b_ref, tthl_ref):
    f32 = jnp.float32
    adj = adj0_ref[...] + adj1_ref[...]
    pres = pres_ref[...]
    itab = itab_ref[...]
    n, d = itab.shape
    ones_col = jnp.ones((n, 1), f32)
    in_deg = jnp.dot(adj, ones_col, preferred_element_type=f32)
    out_deg = jax.lax.dot_general(adj, ones_col, (((0,), (0,)), ((), ())),
                                  preferred_element_type=f32)
    ns = jax.lax.rsqrt(jnp.maximum(out_deg, 1.0))
    nd = jax.lax.rsqrt(jnp.maximum(in_deg, 1.0))

    wh = wh_ref[...]
    whf = whf_ref[...]

    def gnn(hid):
        hd = jnp.dot(hid, wh, preferred_element_type=f32)
        h1, h2, h3 = hd[:, :d], hd[:, d:2 * d], hd[:, 2 * d:]
        agg = nd * jnp.dot(adj, h1 * ns, preferred_element_type=f32)
        hf = jnp.dot(agg, whf, preferred_element_type=f32)
        f1, f2 = hf[:, :d], hf[:, d:]
        return h3 + jnp.maximum(f1 + h2, 0.0) * f2

    hid = itab
    for _ in range(aq_ref.shape[0]):
        hid = gnn(hid)

    dh = d // H
    x = itab
    for i in range(aq_ref.shape[0]):
        q = jnp.dot(x, aq_ref[i], preferred_element_type=f32)
        k = jnp.dot(x, ak_ref[i], preferred_element_type=f32)
        v = jnp.dot(x, av_ref[i], preferred_element_type=f32) * pres
        cols = []
        for h in range(H):
            qh = q[:, h * dh:(h + 1) * dh]
            kh = k[:, h * dh:(h + 1) * dh]
            vh = v[:, h * dh:(h + 1) * dh]
            sc = jnp.tanh(jax.lax.dot_general(
                qh, kh, (((1,), (1,)), ((), ())), preferred_element_type=f32))
            cols.append(jnp.dot(sc, vh, preferred_element_type=f32))
        att = jnp.concatenate(cols, axis=1)
        att = jnp.maximum(jnp.dot(att, afw_ref[i], preferred_element_type=f32)
                          + afb_ref[i:i + 1], 0.0)
        x = x + att

    fw = fw_ref[...]
    t_tab = (jnp.dot(hid, fw[:d], preferred_element_type=f32)
             + jnp.dot(x, fw[d:], preferred_element_type=f32) + fb_ref[...])
    t2_tab = (jnp.dot(t_tab, a2w_ref[...], preferred_element_type=f32)
              + a2b_ref[...])
    t2h = t2_tab.astype(jnp.bfloat16)
    th = t_tab.astype(jnp.bfloat16)
    t2l = (t2_tab - t2h.astype(f32)).astype(jnp.bfloat16)
    tl = (t_tab - th.astype(f32)).astype(jnp.bfloat16)
    tthl_ref[...] = jnp.concatenate([t2h, th, t2l, tl], axis=1)


def _token_body(tok_ref, seq_ref, tthl_ref, a1w_ref, a1b_ref, a3r_ref,
                a4w_ref, a4b_ref, out_ref):
    f32 = jnp.float32
    tb = tok_ref.shape[0]
    n = tthl_ref.shape[0]
    rb, d = out_ref.shape
    ll = tb // rb
    iota_t = jax.lax.broadcasted_iota(jnp.int32, (tb, n), 1)
    oh = (tok_ref[...] == iota_t).astype(jnp.bfloat16)
    g = jnp.dot(oh, tthl_ref[...], preferred_element_type=f32)
    t2g = (g[:, :d] + g[:, 2 * d:3 * d]).reshape(rb, ll, d)
    tg = (g[:, d:2 * d] + g[:, 3 * d:]).reshape(rb, ll, d)
    mask = (tok_ref[...] != 0).astype(f32).reshape(rb, ll)

    lio = jax.lax.broadcasted_iota(jnp.int32, (rb, ll), 1)
    lsel = (lio == (seq_ref[...] - 1)).astype(f32)
    ht = jnp.sum(lsel[:, :, None] * tg, axis=1)
    q1 = jnp.dot(ht, a1w_ref[...], preferred_element_type=f32) + a1b_ref[...]
    sig = jax.nn.sigmoid(q1[:, None, :] + t2g)
    alpha = jnp.sum(sig * a3r_ref[...][None], axis=2)
    w = alpha * mask
    a = jnp.sum(w[:, :, None] * tg, axis=1)
    a4w = a4w_ref[...]
    out_ref[...] = (jnp.dot(a, a4w[:d], preferred_element_type=f32)
                    + jnp.dot(ht, a4w[d:], preferred_element_type=f32)
                    + a4b_ref[...])


def kernel(in_item_id, seqlen, item_table, w_h, w_hf, agnn_q, agnn_k, agnn_v,
           agnn_ffn_w, agnn_ffn_b, fuse_w, fuse_b, att1_w, att1_b, att2_w,
           att2_b, att3_w, att4_w, att4_b):
    f32 = jnp.float32
    b, l = in_item_id.shape
    n, d = item_table.shape
    ids = in_item_id.astype(jnp.int32)
    sl = seqlen.astype(jnp.int32)

    rb = 64
    nb = b // rb
    eb = rb * (l - 1)
    tb = rb * l

    tokf = ids.reshape(-1, 1)
    ids2 = ids.reshape(32, -1, 128)
    idsf = ids.reshape(-1)
    posf = jnp.broadcast_to(jnp.arange(l, dtype=jnp.int32)[None],
                            (b, l)).reshape(-1)
    seqf = jnp.broadcast_to(sl[:, None], (b, l)).reshape(-1)

    mesh = plsc.VectorSubcoreMesh(core_axis_name="c", subcore_axis_name="s")
    sc_graph = functools.partial(
        pl.kernel,
        mesh=mesh,
        out_type=[
            jax.ShapeDtypeStruct((2, _ADJ_PAD), f32),
            jax.ShapeDtypeStruct((2, 1024), f32),
        ],
        scratch_types=[
            pltpu.VMEM((_TOK_W + 16,), jnp.int32),
            pltpu.VMEM((50, 128), jnp.int32),
            pltpu.VMEM((_TOK_W,), jnp.int32),
            pltpu.VMEM((_TOK_W,), jnp.int32),
            pltpu.VMEM((50, 128), jnp.int32),
            pltpu.VMEM((50, 128), jnp.float32),
            pltpu.VMEM((50, 128), jnp.float32),
            pltpu.VMEM((8000,), jnp.float32),
            pltpu.SemaphoreType.DMA,
            pltpu.SemaphoreType.DMA,
            pltpu.VMEM_SHARED((_ADJ_PAD,), jnp.float32),
            pltpu.VMEM_SHARED((1024,), jnp.float32),
        ],
    )(_sc_graph_body)
    adj_parts, pres_parts = sc_graph(ids2, idsf, posf, seqf)

    adj0 = adj_parts[0, :n * n].reshape(n, n)
    adj1 = adj_parts[1, :n * n].reshape(n, n)
    pres_col = ((pres_parts[0, :n] + pres_parts[1, :n]) > 0).astype(
        f32).reshape(n, 1)

    tthl = pl.pallas_call(
        _dense_body,
        out_shape=jax.ShapeDtypeStruct((n, 4 * d), jnp.bfloat16),
        interpret=_interpret,
    )(adj0, adj1, pres_col, item_table, w_h, w_hf, agnn_q, agnn_k, agnn_v,
      agnn_ffn_w, agnn_ffn_b, fuse_w, fuse_b.reshape(1, d),
      att1_w, att1_b.reshape(1, d), att2_w, att2_b.reshape(1, d))

    out = pl.pallas_call(
        _token_body,
        grid=(nb,),
        in_specs=[
            pl.BlockSpec((tb, 1), lambda i: (i, 0)),
            pl.BlockSpec((rb, 1), lambda i: (i, 0)),
            pl.BlockSpec((n, 4 * d), lambda i: (0, 0)),
            pl.BlockSpec((d, d), lambda i: (0, 0)),
            pl.BlockSpec((1, d), lambda i: (0, 0)),
            pl.BlockSpec((1, d), lambda i: (0, 0)),
            pl.BlockSpec((2 * d, d), lambda i: (0, 0)),
            pl.BlockSpec((1, d), lambda i: (0, 0)),
        ],
        out_specs=pl.BlockSpec((rb, d), lambda i: (i, 0)),
        out_shape=jax.ShapeDtypeStruct((b, d), f32),
        interpret=_interpret,
    )(tokf, sl[:, None], tthl, att1_w, att1_b.reshape(1, d),
      att3_w.reshape(1, d), att4_w, att4_b.reshape(1, d))

    return out


# speedup vs baseline: 13.5013x; 1.0321x over previous
import functools

import jax
import jax.numpy as jnp
from jax import lax
from jax.experimental import pallas as pl
from jax.experimental.pallas import tpu as pltpu
from jax.experimental.pallas import tpu_sc as plsc

_interpret = False

H = 4

_NV = 1000
_ADJ_PAD = 1024000
_ROWS_W = 128
_TOK_W = _ROWS_W * 50


def _sc_graph_body(ids_hbm, idsf_hbm, posf_hbm, seqf_hbm, adj_out, pres_out,
                   tokf, tok2, posv, seqv, eidx2, val2, onesb, zbuf,
                   sem_a, sem_p, adj_sp, pres_sp):
    c = lax.axis_index("c")
    s = lax.axis_index("s")
    wid = c * 16 + s
    f32 = jnp.float32
    i32 = jnp.int32

    if True:
        def zfill(i, _):
            zbuf[pl.ds(i * 16, 16)] = jnp.zeros((16,), f32)
            return 0
        lax.fori_loop(0, 500, zfill, 0)
        for m in range(8):
            pltpu.sync_copy(zbuf.at[pl.ds(0, 8000)],
                            adj_sp.at[pl.ds(s * 64000 + m * 8000, 8000)])

        @pl.when(s == 0)
        def _():
            pltpu.sync_copy(zbuf.at[pl.ds(0, 1024)], pres_sp.at[pl.ds(0, 1024)])

        pltpu.sync_copy(ids_hbm.at[wid], tok2)
        pltpu.sync_copy(idsf_hbm.at[pl.ds(wid * _TOK_W, _TOK_W)],
                        tokf.at[pl.ds(0, _TOK_W)])
        pltpu.sync_copy(posf_hbm.at[pl.ds(wid * _TOK_W, _TOK_W)], posv)
        pltpu.sync_copy(seqf_hbm.at[pl.ds(wid * _TOK_W, _TOK_W)], seqv)
        tokf[pl.ds(_TOK_W, 16)] = jnp.zeros((16,), i32)

        def erow(j, _):
            base = j * 128
            for k in range(8):
                off = base + k * 16
                valid = posv[pl.ds(off, 16)] < (seqv[pl.ds(off, 16)] - 1)
                tok_v = tokf[pl.ds(off, 16)]
                tok_n = tokf[pl.ds(off + 1, 16)]
                eidx2[j, pl.ds(k * 16, 16)] = tok_n * _NV + tok_v
                val2[j, pl.ds(k * 16, 16)] = jnp.where(
                    valid, jnp.ones((16,), f32), jnp.zeros((16,), f32))
                onesb[j, pl.ds(k * 16, 16)] = jnp.ones((16,), f32)
            return 0
        lax.fori_loop(0, 50, erow, 0)

        plsc.subcore_barrier()
        adj_descs = [
            pltpu.make_async_copy(val2.at[j], adj_sp.at[eidx2.at[j]], sem_a)
            for j in range(50)
        ]
        pres_descs = [
            pltpu.make_async_copy(onesb.at[j], pres_sp.at[tok2.at[j]], sem_p)
            for j in range(50)
        ]
        for dsc in adj_descs:
            dsc.start(add=True)
        for dsc in pres_descs:
            dsc.start(add=True)
        for dsc in adj_descs:
            dsc.wait()
        for dsc in pres_descs:
            dsc.wait()
        plsc.subcore_barrier()

        pltpu.sync_copy(adj_sp.at[pl.ds(s * 64000, 64000)],
                        adj_out.at[c, pl.ds(s * 64000, 64000)])

        @pl.when(s == 0)
        def _():
            pltpu.sync_copy(pres_sp.at[pl.ds(0, 1024)], pres_out.at[c])


def _dense_body(adj0_ref, adj1_ref, pres_ref, itab_ref, wh_ref, whf_ref, aq_ref, ak_ref,
                av_ref, afw_ref, afb_ref, fw_ref, fb_ref, a1w_ref, a1b_ref,
                a2w_ref, a2b_ref, tthl_ref):
    f32 = jnp.float32
    adj = adj0_ref[...] + adj1_ref[...]
    pres = pres_ref[...]
    itab = itab_ref[...]
    n, d = itab.shape
    ones_col = jnp.ones((n, 1), f32)
    in_deg = jnp.dot(adj, ones_col, preferred_element_type=f32)
    out_deg = jax.lax.dot_general(adj, ones_col, (((0,), (0,)), ((), ())),
                                  preferred_element_type=f32)
    ns = jax.lax.rsqrt(jnp.maximum(out_deg, 1.0))
    nd = jax.lax.rsqrt(jnp.maximum(in_deg, 1.0))

    wh = wh_ref[...]
    whf = whf_ref[...]

    def gnn(hid):
        hd = jnp.dot(hid, wh, preferred_element_type=f32)
        h1, h2, h3 = hd[:, :d], hd[:, d:2 * d], hd[:, 2 * d:]
        agg = nd * jnp.dot(adj, h1 * ns, preferred_element_type=f32)
        hf = jnp.dot(agg, whf, preferred_element_type=f32)
        f1, f2 = hf[:, :d], hf[:, d:]
        return h3 + jnp.maximum(f1 + h2, 0.0) * f2

    hid = itab
    for _ in range(aq_ref.shape[0]):
        hid = gnn(hid)

    dh = d // H
    x = itab
    for i in range(aq_ref.shape[0]):
        q = jnp.dot(x, aq_ref[i], preferred_element_type=f32)
        k = jnp.dot(x, ak_ref[i], preferred_element_type=f32)
        v = jnp.dot(x, av_ref[i], preferred_element_type=f32) * pres
        cols = []
        for h in range(H):
            qh = q[:, h * dh:(h + 1) * dh]
            kh = k[:, h * dh:(h + 1) * dh]
            vh = v[:, h * dh:(h + 1) * dh]
            sc = jnp.tanh(jax.lax.dot_general(
                qh, kh, (((1,), (1,)), ((), ())), preferred_element_type=f32))
            cols.append(jnp.dot(sc, vh, preferred_element_type=f32))
        att = jnp.concatenate(cols, axis=1)
        att = jnp.maximum(jnp.dot(att, afw_ref[i], preferred_element_type=f32)
                          + afb_ref[i:i + 1], 0.0)
        x = x + att

    fw = fw_ref[...]
    t_tab = (jnp.dot(hid, fw[:d], preferred_element_type=f32)
             + jnp.dot(x, fw[d:], preferred_element_type=f32) + fb_ref[...])
    t2_tab = (jnp.dot(t_tab, a2w_ref[...], preferred_element_type=f32)
              + a2b_ref[...])
    tthl_ref[...] = jnp.concatenate(
        [t2_tab.astype(jnp.bfloat16), t_tab.astype(jnp.bfloat16)], axis=1)


def _token_body(tok_ref, seq_ref, tthl_ref, a1w_ref, a1b_ref, a3r_ref,
                a4w_ref, a4b_ref, out_ref):
    f32 = jnp.float32
    tb = tok_ref.shape[0]
    n = tthl_ref.shape[0]
    rb, d = out_ref.shape
    ll = tb // rb
    iota_t = jax.lax.broadcasted_iota(jnp.int32, (tb, n), 1)
    oh = (tok_ref[...] == iota_t).astype(jnp.bfloat16)
    g = jnp.dot(oh, tthl_ref[...], preferred_element_type=f32)
    t2g = g[:, :d].reshape(rb, ll, d)
    tg = g[:, d:].reshape(rb, ll, d)
    mask = (tok_ref[...] != 0).astype(f32).reshape(rb, ll)

    lio = jax.lax.broadcasted_iota(jnp.int32, (rb, ll), 1)
    lsel = (lio == (seq_ref[...] - 1)).astype(f32)
    ht = jnp.sum(lsel[:, :, None] * tg, axis=1)
    q1 = jnp.dot(ht, a1w_ref[...], preferred_element_type=f32) + a1b_ref[...]
    sig = jax.nn.sigmoid(q1[:, None, :] + t2g)
    alpha = jnp.sum(sig * a3r_ref[...][None], axis=2)
    w = alpha * mask
    a = jnp.sum(w[:, :, None] * tg, axis=1)
    a4w = a4w_ref[...]
    out_ref[...] = (jnp.dot(a, a4w[:d], preferred_element_type=f32)
                    + jnp.dot(ht, a4w[d:], preferred_element_type=f32)
                    + a4b_ref[...])


def kernel(in_item_id, seqlen, item_table, w_h, w_hf, agnn_q, agnn_k, agnn_v,
           agnn_ffn_w, agnn_ffn_b, fuse_w, fuse_b, att1_w, att1_b, att2_w,
           att2_b, att3_w, att4_w, att4_b):
    f32 = jnp.float32
    b, l = in_item_id.shape
    n, d = item_table.shape
    ids = in_item_id.astype(jnp.int32)
    sl = seqlen.astype(jnp.int32)

    rb = 64
    nb = b // rb
    eb = rb * (l - 1)
    tb = rb * l

    tokf = ids.reshape(-1, 1)
    ids2 = ids.reshape(32, -1, 128)
    idsf = ids.reshape(-1)
    posf = jnp.broadcast_to(jnp.arange(l, dtype=jnp.int32)[None],
                            (b, l)).reshape(-1)
    seqf = jnp.broadcast_to(sl[:, None], (b, l)).reshape(-1)

    mesh = plsc.VectorSubcoreMesh(core_axis_name="c", subcore_axis_name="s")
    sc_graph = functools.partial(
        pl.kernel,
        mesh=mesh,
        out_type=[
            jax.ShapeDtypeStruct((2, _ADJ_PAD), f32),
            jax.ShapeDtypeStruct((2, 1024), f32),
        ],
        scratch_types=[
            pltpu.VMEM((_TOK_W + 16,), jnp.int32),
            pltpu.VMEM((50, 128), jnp.int32),
            pltpu.VMEM((_TOK_W,), jnp.int32),
            pltpu.VMEM((_TOK_W,), jnp.int32),
            pltpu.VMEM((50, 128), jnp.int32),
            pltpu.VMEM((50, 128), jnp.float32),
            pltpu.VMEM((50, 128), jnp.float32),
            pltpu.VMEM((8000,), jnp.float32),
            pltpu.SemaphoreType.DMA,
            pltpu.SemaphoreType.DMA,
            pltpu.VMEM_SHARED((_ADJ_PAD,), jnp.float32),
            pltpu.VMEM_SHARED((1024,), jnp.float32),
        ],
    )(_sc_graph_body)
    adj_parts, pres_parts = sc_graph(ids2, idsf, posf, seqf)

    adj0 = adj_parts[0, :n * n].reshape(n, n)
    adj1 = adj_parts[1, :n * n].reshape(n, n)
    pres_col = ((pres_parts[0, :n] + pres_parts[1, :n]) > 0).astype(
        f32).reshape(n, 1)

    tthl = pl.pallas_call(
        _dense_body,
        out_shape=jax.ShapeDtypeStruct((n, 2 * d), jnp.bfloat16),
        interpret=_interpret,
    )(adj0, adj1, pres_col, item_table, w_h, w_hf, agnn_q, agnn_k, agnn_v,
      agnn_ffn_w, agnn_ffn_b, fuse_w, fuse_b.reshape(1, d),
      att1_w, att1_b.reshape(1, d), att2_w, att2_b.reshape(1, d))

    out = pl.pallas_call(
        _token_body,
        grid=(nb,),
        in_specs=[
            pl.BlockSpec((tb, 1), lambda i: (i, 0)),
            pl.BlockSpec((rb, 1), lambda i: (i, 0)),
            pl.BlockSpec((n, 2 * d), lambda i: (0, 0)),
            pl.BlockSpec((d, d), lambda i: (0, 0)),
            pl.BlockSpec((1, d), lambda i: (0, 0)),
            pl.BlockSpec((1, d), lambda i: (0, 0)),
            pl.BlockSpec((2 * d, d), lambda i: (0, 0)),
            pl.BlockSpec((1, d), lambda i: (0, 0)),
        ],
        out_specs=pl.BlockSpec((rb, d), lambda i: (i, 0)),
        out_shape=jax.ShapeDtypeStruct((b, d), f32),
        interpret=_interpret,
    )(tokf, sl[:, None], tthl, att1_w, att1_b.reshape(1, d),
      att3_w.reshape(1, d), att4_w, att4_b.reshape(1, d))

    return out
